# baseline (speedup 1.0000x reference)
import functools

import jax
import jax.numpy as jnp
from jax import lax
from jax.experimental import pallas as pl
from jax.experimental.pallas import tpu as pltpu
from jax.experimental.pallas import tpu_sc as plsc

N = 10000
E = 320000
D = 128
H = 256
L = 4
EPS = 1e-7
SHIFT = 12.0
BR = 1000
NB = N // BR

CW = 50
SUP = 40
NSTAGE = E // 16 // (SUP * CW)
W = 4
RING = SUP // W
N_PAD = 10240
DUMP = 624


def _ln(x, g, b, eps=1e-5):
    mu = jnp.mean(x, axis=-1, keepdims=True)
    var = jnp.mean((x - mu) ** 2, axis=-1, keepdims=True)
    return (x - mu) / jnp.sqrt(var + eps) * g + b


def _pre0_body(t_ref, x_ref, m_ref, cmax_ref):
    m = jnp.maximum(x_ref[...], 0.0) + EPS
    m_ref[...] = m
    lg = m * t_ref[0, 0]
    bm = jnp.max(lg, axis=0, keepdims=True)

    @pl.when(pl.program_id(0) == 0)
    def _():
        cmax_ref[...] = jnp.full((1, D), -jnp.inf, jnp.float32)

    cmax_ref[...] = jnp.maximum(cmax_ref[...], bm)


def _tab0_body(t_ref, m_ref, cmax_ref, pq_ref):
    m = m_ref[...]
    p = jnp.exp(m * t_ref[0, 0] - cmax_ref[...])
    pq_ref[0] = p
    pq_ref[1] = m * p


def _dense_body(t_ref, sums_ref, zin_ref, hprev_ref, w1_ref, b1_ref, mg_ref,
                mb_ref, w2_ref, b2_ref, g_ref, bb_ref, h_ref, *rest,
                has_resid, final):
    agg = sums_ref[1] / (sums_ref[0] + 1e-16)
    oc = agg + zin_ref[...]
    y = jnp.dot(oc, w1_ref[...], preferred_element_type=jnp.float32) + b1_ref[...]
    y = jnp.maximum(_ln(y, mg_ref[...], mb_ref[...]), 0.0)
    h = jnp.dot(y, w2_ref[...], preferred_element_type=jnp.float32) + b2_ref[...]
    if has_resid:
        h = h + hprev_ref[...]
    if final:
        h_ref[...] = jnp.maximum(_ln(h, g_ref[...], bb_ref[...]), 0.0)
    else:
        z_ref, pq_ref = rest
        h_ref[...] = h
        z = jnp.maximum(_ln(h, g_ref[...], bb_ref[...]), 0.0)
        z_ref[...] = z
        m = z + EPS
        p = jnp.exp(m * t_ref[0, 0] - SHIFT)
        pq_ref[0] = p
        pq_ref[1] = m * p


def _row_spec(i_map=None):
    return pl.BlockSpec((BR, D), i_map or (lambda i: (i, 0)))


def _full_spec(shape):
    return pl.BlockSpec(shape, lambda i: tuple(0 for _ in shape))


_SMEM_SPEC = pl.BlockSpec(memory_space=pltpu.SMEM)


def _pre0(x, t0):
    return pl.pallas_call(
        _pre0_body,
        grid=(NB,),
        in_specs=[_SMEM_SPEC, _row_spec()],
        out_specs=[_row_spec(), _full_spec((1, D))],
        out_shape=[jax.ShapeDtypeStruct((N, D), jnp.float32),
                   jax.ShapeDtypeStruct((1, D), jnp.float32)],
    )(t0, x)


def _tab0(m, cmax, t0):
    return pl.pallas_call(
        _tab0_body,
        grid=(NB,),
        in_specs=[_SMEM_SPEC, _row_spec(), _full_spec((1, D))],
        out_specs=pl.BlockSpec((2, BR, D), lambda i: (0, i, 0)),
        out_shape=jax.ShapeDtypeStruct((2, N, D), jnp.float32),
    )(t0, m, cmax)


def _dense(sums, zin, hprev, w1, b1, mg, mb, w2, b2, g, bb, tn,
           has_resid, final):
    body = functools.partial(_dense_body, has_resid=has_resid, final=final)
    out_specs = [_row_spec()]
    out_shape = [jax.ShapeDtypeStruct((N, D), jnp.float32)]
    if not final:
        out_specs += [_row_spec(), pl.BlockSpec((2, BR, D), lambda i: (0, i, 0))]
        out_shape += [jax.ShapeDtypeStruct((N, D), jnp.float32),
                      jax.ShapeDtypeStruct((2, N, D), jnp.float32)]
    return pl.pallas_call(
        body,
        grid=(NB,),
        in_specs=[_SMEM_SPEC,
                  pl.BlockSpec((2, BR, D), lambda i: (0, i, 0)),
                  _row_spec(),
                  _row_spec(),
                  _full_spec((D, H)), _full_spec((1, H)),
                  _full_spec((1, H)), _full_spec((1, H)),
                  _full_spec((H, D)), _full_spec((1, D)),
                  _full_spec((1, D)), _full_spec((1, D))],
        out_specs=out_specs,
        out_shape=out_shape,
    )(tn, sums, zin, hprev, w1, b1, mg, mb, w2, b2, g, bb)


def _sc_body(tab, idxall, out, ib0, ib1, r0, r1, r2, r3, acc,
             i0, i1, g0, g1, g2, g3, s0, s1, s2, s3):
    ibs = [ib0, ib1]
    isems = [i0, i1]
    rows = [r0, r1, r2, r3]
    gs = [g0, g1, g2, g3]
    ss = [s0, s1, s2, s3]
    c = lax.axis_index("c")
    s = lax.axis_index("s")

    def _iload(st, b):
        base = pl.multiple_of(((c * 16 + s) * NSTAGE + st) * 2 * SUP, 8)
        pltpu.async_copy(idxall.at[pl.ds(base, 2 * SUP)], ibs[b], isems[b])

    _iload(0, 0)

    def _zrow(r, _):
        for k in range(D // 16):
            r0[r, pl.ds(k * 16, 16)] = jnp.zeros((16,), jnp.float32)
        return 0

    lax.fori_loop(0, 40, _zrow, 0)

    def _zcp(j, _):
        pltpu.sync_copy(r0.at[pl.ds(0, 40)],
                        acc.at[pl.ds(pl.multiple_of(s * 640, 8) + j * 40, 40)])
        return 0

    lax.fori_loop(0, 16, _zcp, 0)
    plsc.subcore_barrier()

    def _stage(st, ibsel):
        ib = ibs[ibsel]
        pltpu.make_async_copy(idxall.at[pl.ds(0, 2 * SUP)], ib, isems[ibsel]).wait()

        @pl.when(st < NSTAGE - 1)
        def _():
            _iload(st + 1, 1 - ibsel)

        for w in range(W):
            pltpu.async_copy(tab.at[ib.at[w]], rows[w], gs[w])

        def _ring(g, _):
            for w in range(W):
                i = W * g + w
                pltpu.make_async_copy(tab.at[ib.at[i]], rows[w], gs[w]).wait()
                pltpu.async_copy(rows[w], acc.at[ib.at[SUP + i]], ss[w], add=True)
                pw = (w - 1) % W
                cond = (g > 0) if w == 0 else (g < RING - 1)

                @pl.when(cond)
                def _():
                    pltpu.make_async_copy(rows[pw], acc.at[ib.at[SUP + i - 1]], ss[pw]).wait()
                    pltpu.async_copy(tab.at[ib.at[i - 1 + W]], rows[pw], gs[pw])
            return 0

        lax.fori_loop(0, RING, _ring, 0)
        for w in range(W):
            pltpu.make_async_copy(rows[w], acc.at[ib.at[2 * SUP - W + w]], ss[w]).wait()

    def _pair(k, _):
        _stage(2 * k, 0)
        _stage(2 * k + 1, 1)
        return 0

    lax.fori_loop(0, NSTAGE // 2, _pair, 0)
    if NSTAGE % 2:
        _stage(jnp.int32(NSTAGE - 1), 0)
    plsc.subcore_barrier()

    pltpu.sync_copy(acc.at[pl.ds(pl.multiple_of(s * DUMP, 8), DUMP)],
                    out.at[pl.ds(pl.multiple_of(c * N + s * DUMP, 8), DUMP)])

    @pl.when(s == 15)
    def _():
        pltpu.sync_copy(acc.at[pl.ds(16 * DUMP, 16)],
                        out.at[pl.ds(pl.multiple_of(c * N, 8) + 16 * DUMP, 16)])


def _sc_scatter(tab, idxall):
    mesh = plsc.VectorSubcoreMesh(core_axis_name="c", subcore_axis_name="s")
    return pl.kernel(
        _sc_body,
        mesh=mesh,
        out_type=jax.ShapeDtypeStruct((2 * N, D), jnp.float32),
        scratch_types=(
            [pltpu.VMEM((2 * SUP, CW), jnp.int32)] * 2
            + [pltpu.VMEM((CW, D), jnp.float32)] * W
            + [pltpu.VMEM_SHARED((N_PAD, D), jnp.float32)]
            + [pltpu.SemaphoreType.DMA] * (2 + 2 * W)
        ),
    )(tab, idxall)


def kernel(x, edge_index, t, W1, b1, mg, mb, W2, b2, lng, lnb):
    src = edge_index[0]
    dst = edge_index[1]
    sv = src.reshape(16, NSTAGE, SUP, CW)
    dv = dst.reshape(16, NSTAGE, SUP, CW)
    idxall = jnp.stack([
        jnp.concatenate([sv, dv], axis=2),
        jnp.concatenate([sv + N, dv], axis=2),
    ]).reshape(-1, CW)
    del sv, dv
    ts = t.reshape(L, 1, 1)

    m0, cmax = _pre0(x, ts[0])
    pq = _tab0(m0, cmax, ts[0])

    h = None
    zin = x
    for l in range(L):
        sums = _sc_scatter(pq.reshape(2 * N, D), idxall).reshape(2, N, D)
        final = l == L - 1
        g_idx = 0 if final else l + 1
        tn = ts[0] if final else ts[l + 1]
        outs = _dense(sums, zin, x if h is None else h,
                      W1[l], b1[l].reshape(1, H), mg[l].reshape(1, H),
                      mb[l].reshape(1, H), W2[l], b2[l].reshape(1, D),
                      lng[g_idx].reshape(1, D), lnb[g_idx].reshape(1, D),
                      tn, has_resid=l > 0, final=final)
        if final:
            return outs[0]
        h, zin, pq = outs


# speedup vs baseline: 1.0816x; 1.0816x over previous
import functools

import jax
import jax.numpy as jnp
from jax import lax
from jax.experimental import pallas as pl
from jax.experimental.pallas import tpu as pltpu
from jax.experimental.pallas import tpu_sc as plsc

N = 10000
E = 320000
D = 128
H = 256
L = 4
EPS = 1e-7
SHIFT = 12.0
BR = 1000
NB = N // BR

CW = 50
SUP = 40
NSTAGE = E // 16 // (SUP * CW)
W = 4
RING = SUP // W
N_PAD = 10240
DUMP = 624


def _ln(x, g, b, eps=1e-5):
    mu = jnp.mean(x, axis=-1, keepdims=True)
    var = jnp.mean((x - mu) ** 2, axis=-1, keepdims=True)
    return (x - mu) / jnp.sqrt(var + eps) * g + b


def _pre0_body(t_ref, x_ref, m_ref, cmax_ref):
    m = jnp.maximum(x_ref[...], 0.0) + EPS
    m_ref[...] = m
    lg = m * t_ref[0, 0]
    bm = jnp.max(lg, axis=0, keepdims=True)

    @pl.when(pl.program_id(0) == 0)
    def _():
        cmax_ref[...] = jnp.full((1, D), -jnp.inf, jnp.float32)

    cmax_ref[...] = jnp.maximum(cmax_ref[...], bm)


def _tab0_body(t_ref, m_ref, cmax_ref, pq_ref):
    m = m_ref[...]
    p = jnp.exp(m * t_ref[0, 0] - cmax_ref[...])
    pq_ref[0] = p
    pq_ref[1] = m * p


def _dense_body(t_ref, sums_ref, zin_ref, hprev_ref, w1_ref, b1_ref, mg_ref,
                mb_ref, w2_ref, b2_ref, g_ref, bb_ref, h_ref, *rest,
                has_resid, final):
    agg = sums_ref[1] / (sums_ref[0] + 1e-16)
    oc = agg + zin_ref[...]
    y = jnp.dot(oc, w1_ref[...], preferred_element_type=jnp.float32) + b1_ref[...]
    y = jnp.maximum(_ln(y, mg_ref[...], mb_ref[...]), 0.0)
    h = jnp.dot(y, w2_ref[...], preferred_element_type=jnp.float32) + b2_ref[...]
    if has_resid:
        h = h + hprev_ref[...]
    if final:
        h_ref[...] = jnp.maximum(_ln(h, g_ref[...], bb_ref[...]), 0.0)
    else:
        z_ref, pq_ref = rest
        h_ref[...] = h
        z = jnp.maximum(_ln(h, g_ref[...], bb_ref[...]), 0.0)
        z_ref[...] = z
        m = z + EPS
        p = jnp.exp(m * t_ref[0, 0] - SHIFT)
        pq_ref[0] = p
        pq_ref[1] = m * p


def _row_spec(i_map=None):
    return pl.BlockSpec((BR, D), i_map or (lambda i: (i, 0)))


def _full_spec(shape):
    return pl.BlockSpec(shape, lambda i: tuple(0 for _ in shape))


_SMEM_SPEC = pl.BlockSpec(memory_space=pltpu.SMEM)


def _pre0(x, t0):
    return pl.pallas_call(
        _pre0_body,
        grid=(NB,),
        in_specs=[_SMEM_SPEC, _row_spec()],
        out_specs=[_row_spec(), _full_spec((1, D))],
        out_shape=[jax.ShapeDtypeStruct((N, D), jnp.float32),
                   jax.ShapeDtypeStruct((1, D), jnp.float32)],
    )(t0, x)


def _tab0(m, cmax, t0):
    return pl.pallas_call(
        _tab0_body,
        grid=(NB,),
        in_specs=[_SMEM_SPEC, _row_spec(), _full_spec((1, D))],
        out_specs=pl.BlockSpec((2, BR, D), lambda i: (0, i, 0)),
        out_shape=jax.ShapeDtypeStruct((2, N, D), jnp.float32),
    )(t0, m, cmax)


def _dense(sums, zin, hprev, w1, b1, mg, mb, w2, b2, g, bb, tn,
           has_resid, final):
    body = functools.partial(_dense_body, has_resid=has_resid, final=final)
    out_specs = [_row_spec()]
    out_shape = [jax.ShapeDtypeStruct((N, D), jnp.float32)]
    if not final:
        out_specs += [_row_spec(), pl.BlockSpec((2, BR, D), lambda i: (0, i, 0))]
        out_shape += [jax.ShapeDtypeStruct((N, D), jnp.float32),
                      jax.ShapeDtypeStruct((2, N, D), jnp.float32)]
    return pl.pallas_call(
        body,
        grid=(NB,),
        in_specs=[_SMEM_SPEC,
                  pl.BlockSpec((2, BR, D), lambda i: (0, i, 0)),
                  _row_spec(),
                  _row_spec(),
                  _full_spec((D, H)), _full_spec((1, H)),
                  _full_spec((1, H)), _full_spec((1, H)),
                  _full_spec((H, D)), _full_spec((1, D)),
                  _full_spec((1, D)), _full_spec((1, D))],
        out_specs=out_specs,
        out_shape=out_shape,
    )(tn, sums, zin, hprev, w1, b1, mg, mb, w2, b2, g, bb)


def _sc_body(tab, idxall, out, ib0, ib1, r0, r1, r2, r3, acc,
             i0, i1, g0, g1, g2, g3, s0, s1, s2, s3):
    ibs = [ib0, ib1]
    isems = [i0, i1]
    rows = [r0, r1, r2, r3]
    gs = [g0, g1, g2, g3]
    ss = [s0, s1, s2, s3]
    c = lax.axis_index("c")
    s = lax.axis_index("s")

    def _iload(st, b):
        base = pl.multiple_of(((c * 16 + s) * NSTAGE + st) * 2 * SUP, 8)
        pltpu.async_copy(idxall.at[pl.ds(base, 2 * SUP)], ibs[b], isems[b])

    _iload(0, 0)

    def _zrow(r, _):
        for k in range(D // 16):
            r0[r, pl.ds(k * 16, 16)] = jnp.zeros((16,), jnp.float32)
        return 0

    lax.fori_loop(0, 40, _zrow, 0)

    def _zcp(j, _):
        pltpu.sync_copy(r0.at[pl.ds(0, 40)],
                        acc.at[pl.ds(pl.multiple_of(s * 640, 8) + j * 40, 40)])
        return 0

    lax.fori_loop(0, 16, _zcp, 0)
    plsc.subcore_barrier()

    def _stage(st, ibsel):
        ib = ibs[ibsel]
        pltpu.make_async_copy(idxall.at[pl.ds(0, 2 * SUP)], ib, isems[ibsel]).wait()

        @pl.when(st < NSTAGE - 1)
        def _():
            _iload(st + 1, 1 - ibsel)

        for w in range(W):
            pltpu.async_copy(tab.at[ib.at[w]], rows[w], gs[w])

        def _ring(g, _):
            for w in range(W):
                i = W * g + w
                pltpu.make_async_copy(tab.at[ib.at[i]], rows[w], gs[w]).wait()
                pltpu.async_copy(rows[w], acc.at[ib.at[SUP + i]], ss[w], add=True)

                @pl.when(g < RING - 1)
                def _():
                    pltpu.make_async_copy(rows[w], acc.at[ib.at[SUP + i]], ss[w]).wait()
                    pltpu.async_copy(tab.at[ib.at[i + W]], rows[w], gs[w])
            return 0

        lax.fori_loop(0, RING, _ring, 0)
        for w in range(W):
            pltpu.make_async_copy(rows[w], acc.at[ib.at[2 * SUP - W + w]], ss[w]).wait()

    def _pair(k, _):
        _stage(2 * k, 0)
        _stage(2 * k + 1, 1)
        return 0

    lax.fori_loop(0, NSTAGE // 2, _pair, 0)
    if NSTAGE % 2:
        _stage(jnp.int32(NSTAGE - 1), 0)
    plsc.subcore_barrier()

    pltpu.sync_copy(acc.at[pl.ds(pl.multiple_of(s * DUMP, 8), DUMP)],
                    out.at[pl.ds(pl.multiple_of(c * N + s * DUMP, 8), DUMP)])

    @pl.when(s == 15)
    def _():
        pltpu.sync_copy(acc.at[pl.ds(16 * DUMP, 16)],
                        out.at[pl.ds(pl.multiple_of(c * N, 8) + 16 * DUMP, 16)])


def _sc_scatter(tab, idxall):
    mesh = plsc.VectorSubcoreMesh(core_axis_name="c", subcore_axis_name="s")
    return pl.kernel(
        _sc_body,
        mesh=mesh,
        out_type=jax.ShapeDtypeStruct((2 * N, D), jnp.float32),
        scratch_types=(
            [pltpu.VMEM((2 * SUP, CW), jnp.int32)] * 2
            + [pltpu.VMEM((CW, D), jnp.float32)] * W
            + [pltpu.VMEM_SHARED((N_PAD, D), jnp.float32)]
            + [pltpu.SemaphoreType.DMA] * (2 + 2 * W)
        ),
    )(tab, idxall)


def kernel(x, edge_index, t, W1, b1, mg, mb, W2, b2, lng, lnb):
    src = edge_index[0]
    dst = edge_index[1]
    sv = src.reshape(16, NSTAGE, SUP, CW)
    dv = dst.reshape(16, NSTAGE, SUP, CW)
    idxall = jnp.stack([
        jnp.concatenate([sv, dv], axis=2),
        jnp.concatenate([sv + N, dv], axis=2),
    ]).reshape(-1, CW)
    del sv, dv
    ts = t.reshape(L, 1, 1)

    m0, cmax = _pre0(x, ts[0])
    pq = _tab0(m0, cmax, ts[0])

    h = None
    zin = x
    for l in range(L):
        sums = _sc_scatter(pq.reshape(2 * N, D), idxall).reshape(2, N, D)
        final = l == L - 1
        g_idx = 0 if final else l + 1
        tn = ts[0] if final else ts[l + 1]
        outs = _dense(sums, zin, x if h is None else h,
                      W1[l], b1[l].reshape(1, H), mg[l].reshape(1, H),
                      mb[l].reshape(1, H), W2[l], b2[l].reshape(1, D),
                      lng[g_idx].reshape(1, D), lnb[g_idx].reshape(1, D),
                      tn, has_resid=l > 0, final=final)
        if final:
            return outs[0]
        h, zin, pq = outs


# speedup vs baseline: 1.0879x; 1.0058x over previous
import functools

import jax
import jax.numpy as jnp
from jax import lax
from jax.experimental import pallas as pl
from jax.experimental.pallas import tpu as pltpu
from jax.experimental.pallas import tpu_sc as plsc

N = 10000
E = 320000
D = 128
H = 256
L = 4
EPS = 1e-7
SHIFT = 12.0
BR = 1000
NB = N // BR

CW = 50
SUP = 40
NSTAGE = E // 16 // (SUP * CW)
W = 4
RING = SUP // W
N_PAD = 10240
DUMP = 624


def _ln(x, g, b, eps=1e-5):
    mu = jnp.mean(x, axis=-1, keepdims=True)
    var = jnp.mean((x - mu) ** 2, axis=-1, keepdims=True)
    return (x - mu) / jnp.sqrt(var + eps) * g + b


def _tab0_body(t_ref, x_ref, pq_ref, m_scr, cmax_scr):
    i = pl.program_id(0)

    @pl.when(i < NB)
    def _():
        m = jnp.maximum(x_ref[...], 0.0) + EPS
        m_scr[pl.ds(i * BR, BR), :] = m
        bm = jnp.max(m * t_ref[0, 0], axis=0, keepdims=True)
        prev = jnp.where(i == 0, jnp.full((1, D), -jnp.inf, jnp.float32),
                         cmax_scr[...])
        cmax_scr[...] = jnp.maximum(prev, bm)

    @pl.when(i >= NB)
    def _():
        m = m_scr[pl.ds((i - NB) * BR, BR), :]
        p = jnp.exp(m * t_ref[0, 0] - cmax_scr[...])
        pq_ref[0] = p
        pq_ref[1] = m * p


def _dense_body(t_ref, sums_ref, zin_ref, hprev_ref, w1_ref, b1_ref, mg_ref,
                mb_ref, w2_ref, b2_ref, g_ref, bb_ref, h_ref, *rest,
                has_resid, final):
    agg = sums_ref[1] / (sums_ref[0] + 1e-16)
    oc = agg + zin_ref[...]
    y = jnp.dot(oc, w1_ref[...], preferred_element_type=jnp.float32) + b1_ref[...]
    y = jnp.maximum(_ln(y, mg_ref[...], mb_ref[...]), 0.0)
    h = jnp.dot(y, w2_ref[...], preferred_element_type=jnp.float32) + b2_ref[...]
    if has_resid:
        h = h + hprev_ref[...]
    if final:
        h_ref[...] = jnp.maximum(_ln(h, g_ref[...], bb_ref[...]), 0.0)
    else:
        z_ref, pq_ref = rest
        h_ref[...] = h
        z = jnp.maximum(_ln(h, g_ref[...], bb_ref[...]), 0.0)
        z_ref[...] = z
        m = z + EPS
        p = jnp.exp(m * t_ref[0, 0] - SHIFT)
        pq_ref[0] = p
        pq_ref[1] = m * p


def _row_spec(i_map=None):
    return pl.BlockSpec((BR, D), i_map or (lambda i: (i, 0)))


def _full_spec(shape):
    return pl.BlockSpec(shape, lambda i: tuple(0 for _ in shape))


_SMEM_SPEC = pl.BlockSpec(memory_space=pltpu.SMEM)


def _tab0(x, t0):
    return pl.pallas_call(
        _tab0_body,
        grid=(2 * NB,),
        in_specs=[_SMEM_SPEC,
                  pl.BlockSpec((BR, D), lambda i: (jnp.where(i < NB, i, i - NB), 0))],
        out_specs=pl.BlockSpec((2, BR, D),
                               lambda i: (0, jnp.where(i < NB, 0, i - NB), 0)),
        out_shape=jax.ShapeDtypeStruct((2, N, D), jnp.float32),
        scratch_shapes=[pltpu.VMEM((N, D), jnp.float32),
                        pltpu.VMEM((1, D), jnp.float32)],
    )(t0, x)


def _dense(sums, zin, hprev, w1, b1, mg, mb, w2, b2, g, bb, tn,
           has_resid, final):
    body = functools.partial(_dense_body, has_resid=has_resid, final=final)
    out_specs = [_row_spec()]
    out_shape = [jax.ShapeDtypeStruct((N, D), jnp.float32)]
    if not final:
        out_specs += [_row_spec(), pl.BlockSpec((2, BR, D), lambda i: (0, i, 0))]
        out_shape += [jax.ShapeDtypeStruct((N, D), jnp.float32),
                      jax.ShapeDtypeStruct((2, N, D), jnp.float32)]
    return pl.pallas_call(
        body,
        grid=(NB,),
        in_specs=[_SMEM_SPEC,
                  pl.BlockSpec((2, BR, D), lambda i: (0, i, 0)),
                  _row_spec(),
                  _row_spec(),
                  _full_spec((D, H)), _full_spec((1, H)),
                  _full_spec((1, H)), _full_spec((1, H)),
                  _full_spec((H, D)), _full_spec((1, D)),
                  _full_spec((1, D)), _full_spec((1, D))],
        out_specs=out_specs,
        out_shape=out_shape,
    )(tn, sums, zin, hprev, w1, b1, mg, mb, w2, b2, g, bb)


def _sc_body(tab, idxall, out, ib0, ib1, r0, r1, r2, r3, acc,
             i0, i1, g0, g1, g2, g3, s0, s1, s2, s3):
    ibs = [ib0, ib1]
    isems = [i0, i1]
    rows = [r0, r1, r2, r3]
    gs = [g0, g1, g2, g3]
    ss = [s0, s1, s2, s3]
    c = lax.axis_index("c")
    s = lax.axis_index("s")

    def _iload(st, b):
        base = pl.multiple_of(((c * 16 + s) * NSTAGE + st) * 2 * SUP, 8)
        pltpu.async_copy(idxall.at[pl.ds(base, 2 * SUP)], ibs[b], isems[b])

    _iload(0, 0)

    def _zrow(r, _):
        for k in range(D // 16):
            r0[r, pl.ds(k * 16, 16)] = jnp.zeros((16,), jnp.float32)
        return 0

    lax.fori_loop(0, 40, _zrow, 0)

    def _zcp(j, _):
        pltpu.sync_copy(r0.at[pl.ds(0, 40)],
                        acc.at[pl.ds(pl.multiple_of(s * 640, 8) + j * 40, 40)])
        return 0

    lax.fori_loop(0, 16, _zcp, 0)
    plsc.subcore_barrier()

    def _stage(st, ibsel):
        ib = ibs[ibsel]
        pltpu.make_async_copy(idxall.at[pl.ds(0, 2 * SUP)], ib, isems[ibsel]).wait()

        @pl.when(st < NSTAGE - 1)
        def _():
            _iload(st + 1, 1 - ibsel)

        for w in range(W):
            pltpu.async_copy(tab.at[ib.at[w]], rows[w], gs[w])

        def _ring(g, _):
            for w in range(W):
                i = W * g + w
                pltpu.make_async_copy(tab.at[ib.at[i]], rows[w], gs[w]).wait()
                pltpu.async_copy(rows[w], acc.at[ib.at[SUP + i]], ss[w], add=True)

                @pl.when(g < RING - 1)
                def _():
                    pltpu.make_async_copy(rows[w], acc.at[ib.at[SUP + i]], ss[w]).wait()
                    pltpu.async_copy(tab.at[ib.at[i + W]], rows[w], gs[w])
            return 0

        lax.fori_loop(0, RING, _ring, 0)
        for w in range(W):
            pltpu.make_async_copy(rows[w], acc.at[ib.at[2 * SUP - W + w]], ss[w]).wait()

    def _pair(k, _):
        _stage(2 * k, 0)
        _stage(2 * k + 1, 1)
        return 0

    lax.fori_loop(0, NSTAGE // 2, _pair, 0)
    if NSTAGE % 2:
        _stage(jnp.int32(NSTAGE - 1), 0)
    plsc.subcore_barrier()

    pltpu.sync_copy(acc.at[pl.ds(pl.multiple_of(s * DUMP, 8), DUMP)],
                    out.at[pl.ds(pl.multiple_of(c * N + s * DUMP, 8), DUMP)])

    @pl.when(s == 15)
    def _():
        pltpu.sync_copy(acc.at[pl.ds(16 * DUMP, 16)],
                        out.at[pl.ds(pl.multiple_of(c * N, 8) + 16 * DUMP, 16)])


def _sc_scatter(tab, idxall):
    mesh = plsc.VectorSubcoreMesh(core_axis_name="c", subcore_axis_name="s")
    return pl.kernel(
        _sc_body,
        mesh=mesh,
        out_type=jax.ShapeDtypeStruct((2 * N, D), jnp.float32),
        scratch_types=(
            [pltpu.VMEM((2 * SUP, CW), jnp.int32)] * 2
            + [pltpu.VMEM((CW, D), jnp.float32)] * W
            + [pltpu.VMEM_SHARED((N_PAD, D), jnp.float32)]
            + [pltpu.SemaphoreType.DMA] * (2 + 2 * W)
        ),
    )(tab, idxall)


def kernel(x, edge_index, t, W1, b1, mg, mb, W2, b2, lng, lnb):
    src = edge_index[0]
    dst = edge_index[1]
    sv = src.reshape(16, NSTAGE, SUP, CW)
    dv = dst.reshape(16, NSTAGE, SUP, CW)
    idxall = jnp.stack([
        jnp.concatenate([sv, dv], axis=2),
        jnp.concatenate([sv + N, dv], axis=2),
    ]).reshape(-1, CW)
    del sv, dv
    ts = t.reshape(L, 1, 1)

    pq = _tab0(x, ts[0])

    h = None
    zin = x
    for l in range(L):
        sums = _sc_scatter(pq.reshape(2 * N, D), idxall).reshape(2, N, D)
        final = l == L - 1
        g_idx = 0 if final else l + 1
        tn = ts[0] if final else ts[l + 1]
        outs = _dense(sums, zin, x if h is None else h,
                      W1[l], b1[l].reshape(1, H), mg[l].reshape(1, H),
                      mb[l].reshape(1, H), W2[l], b2[l].reshape(1, D),
                      lng[g_idx].reshape(1, D), lnb[g_idx].reshape(1, D),
                      tn, has_resid=l > 0, final=final)
        if final:
            return outs[0]
        h, zin, pq = outs


# speedup vs baseline: 1.0989x; 1.0101x over previous
import functools

import jax
import jax.numpy as jnp
from jax import lax
from jax.experimental import pallas as pl
from jax.experimental.pallas import tpu as pltpu
from jax.experimental.pallas import tpu_sc as plsc

N = 10000
E = 320000
D = 128
H = 256
L = 4
EPS = 1e-7
SHIFT = 12.0
BR = 1000
NB = N // BR

CW = 50
SUP = 40
NSTAGE = E // 16 // (SUP * CW)
W = 4
RING = SUP // W
N_PAD = 10240
DUMP = 624


def _ln(x, g, b, eps=1e-5):
    mu = jnp.mean(x, axis=-1, keepdims=True)
    var = jnp.mean((x - mu) ** 2, axis=-1, keepdims=True)
    return (x - mu) / jnp.sqrt(var + eps) * g + b


def _tab0_body(t_ref, x_ref, pq_ref, m_scr, cmax_scr):
    i = pl.program_id(0)

    @pl.when(i < NB)
    def _():
        m = jnp.maximum(x_ref[...], 0.0) + EPS
        m_scr[pl.ds(i * BR, BR), :] = m
        bm = jnp.max(m * t_ref[0, 0], axis=0, keepdims=True)
        prev = jnp.where(i == 0, jnp.full((1, D), -jnp.inf, jnp.float32),
                         cmax_scr[...])
        cmax_scr[...] = jnp.maximum(prev, bm)

    @pl.when(i >= NB)
    def _():
        m = m_scr[pl.ds((i - NB) * BR, BR), :]
        p = jnp.exp(m * t_ref[0, 0] - cmax_scr[...])
        pq_ref[0] = p
        pq_ref[1] = m * p


def _dense_body(t_ref, sums_ref, zin_ref, hprev_ref, w1_ref, b1_ref, mg_ref,
                mb_ref, w2_ref, b2_ref, g_ref, bb_ref, h_ref, *rest,
                has_resid, final):
    agg = sums_ref[1] / (sums_ref[0] + 1e-16)
    oc = agg + zin_ref[...]
    y = jnp.dot(oc, w1_ref[...], preferred_element_type=jnp.float32) + b1_ref[...]
    y = jnp.maximum(_ln(y, mg_ref[...], mb_ref[...]), 0.0)
    h = jnp.dot(y, w2_ref[...], preferred_element_type=jnp.float32) + b2_ref[...]
    if has_resid:
        h = h + hprev_ref[...]
    if final:
        h_ref[...] = jnp.maximum(_ln(h, g_ref[...], bb_ref[...]), 0.0)
    else:
        z_ref, pq_ref = rest
        h_ref[...] = h
        z = jnp.maximum(_ln(h, g_ref[...], bb_ref[...]), 0.0)
        z_ref[...] = z
        m = z + EPS
        p = jnp.exp(m * t_ref[0, 0] - SHIFT)
        pq_ref[0] = p
        pq_ref[1] = m * p


def _row_spec(i_map=None):
    return pl.BlockSpec((BR, D), i_map or (lambda i: (i, 0)))


def _full_spec(shape):
    return pl.BlockSpec(shape, lambda i: tuple(0 for _ in shape))


_SMEM_SPEC = pl.BlockSpec(memory_space=pltpu.SMEM)


def _tab0(x, t0):
    return pl.pallas_call(
        _tab0_body,
        grid=(2 * NB,),
        in_specs=[_SMEM_SPEC,
                  pl.BlockSpec((BR, D), lambda i: (jnp.where(i < NB, i, i - NB), 0))],
        out_specs=pl.BlockSpec((2, BR, D),
                               lambda i: (0, jnp.where(i < NB, 0, i - NB), 0)),
        out_shape=jax.ShapeDtypeStruct((2, N, D), jnp.float32),
        scratch_shapes=[pltpu.VMEM((N, D), jnp.float32),
                        pltpu.VMEM((1, D), jnp.float32)],
    )(t0, x)


def _dense(sums, zin, hprev, w1, b1, mg, mb, w2, b2, g, bb, tn,
           has_resid, final):
    body = functools.partial(_dense_body, has_resid=has_resid, final=final)
    out_specs = [_row_spec()]
    out_shape = [jax.ShapeDtypeStruct((N, D), jnp.float32)]
    if not final:
        out_specs += [_row_spec(), pl.BlockSpec((2, BR, D), lambda i: (0, i, 0))]
        out_shape += [jax.ShapeDtypeStruct((N, D), jnp.float32),
                      jax.ShapeDtypeStruct((2, N, D), jnp.float32)]
    return pl.pallas_call(
        body,
        grid=(NB,),
        in_specs=[_SMEM_SPEC,
                  pl.BlockSpec((2, BR, D), lambda i: (0, i, 0)),
                  _row_spec(),
                  _row_spec(),
                  _full_spec((D, H)), _full_spec((1, H)),
                  _full_spec((1, H)), _full_spec((1, H)),
                  _full_spec((H, D)), _full_spec((1, D)),
                  _full_spec((1, D)), _full_spec((1, D))],
        out_specs=out_specs,
        out_shape=out_shape,
    )(tn, sums, zin, hprev, w1, b1, mg, mb, w2, b2, g, bb)


def _sc_body(tab, idxall, out, ib0, ib1, r0, r1, r2, r3, acc,
             i0, i1, g0, g1, g2, g3, s0, s1, s2, s3):
    ibs = [ib0, ib1]
    isems = [i0, i1]
    rows = [r0, r1, r2, r3]
    gs = [g0, g1, g2, g3]
    ss = [s0, s1, s2, s3]
    c = lax.axis_index("c")
    s = lax.axis_index("s")

    def _iload(st, b):
        base = pl.multiple_of(((c * 16 + s) * NSTAGE + st) * 2 * SUP, 8)
        pltpu.async_copy(idxall.at[pl.ds(base, 2 * SUP)], ibs[b], isems[b])

    _iload(0, 0)

    def _zrow(r, _):
        for k in range(D // 16):
            r0[r, pl.ds(k * 16, 16)] = jnp.zeros((16,), jnp.float32)
        return 0

    lax.fori_loop(0, 40, _zrow, 0)

    def _zcp(j, _):
        pltpu.sync_copy(r0.at[pl.ds(0, 40)],
                        acc.at[pl.ds(pl.multiple_of(s * 640, 8) + j * 40, 40)])
        return 0

    lax.fori_loop(0, 16, _zcp, 0)
    plsc.subcore_barrier()

    def _stage(st, ibsel):
        ib = ibs[ibsel]
        nib = ibs[1 - ibsel]

        def _ring(g, _):
            for w in range(W):
                i = W * g + w
                pltpu.make_async_copy(tab.at[ib.at[i]], rows[w], gs[w]).wait()
                pltpu.async_copy(rows[w], acc.at[ib.at[SUP + i]], ss[w], add=True)

                @pl.when(g < RING - 1)
                def _():
                    pltpu.make_async_copy(rows[w], acc.at[ib.at[SUP + i]], ss[w]).wait()
                    pltpu.async_copy(tab.at[ib.at[i + W]], rows[w], gs[w])
            return 0

        lax.fori_loop(0, RING, _ring, 0)

        @pl.when(st < NSTAGE - 1)
        def _():
            pltpu.make_async_copy(idxall.at[pl.ds(0, 2 * SUP)], nib,
                                  isems[1 - ibsel]).wait()
            for w in range(W):
                pltpu.make_async_copy(rows[w], acc.at[ib.at[2 * SUP - W + w]], ss[w]).wait()
                pltpu.async_copy(tab.at[nib.at[w]], rows[w], gs[w])

            @pl.when(st < NSTAGE - 2)
            def _():
                _iload(st + 2, ibsel)

        @pl.when(st == NSTAGE - 1)
        def _():
            for w in range(W):
                pltpu.make_async_copy(rows[w], acc.at[ib.at[2 * SUP - W + w]], ss[w]).wait()

    pltpu.make_async_copy(idxall.at[pl.ds(0, 2 * SUP)], ib0, isems[0]).wait()
    for w in range(W):
        pltpu.async_copy(tab.at[ib0.at[w]], rows[w], gs[w])
    if NSTAGE > 1:
        _iload(1, 1)

    def _pair(k, _):
        _stage(2 * k, 0)
        _stage(2 * k + 1, 1)
        return 0

    lax.fori_loop(0, NSTAGE // 2, _pair, 0)
    if NSTAGE % 2:
        _stage(jnp.int32(NSTAGE - 1), 0)
    plsc.subcore_barrier()

    pltpu.sync_copy(acc.at[pl.ds(pl.multiple_of(s * DUMP, 8), DUMP)],
                    out.at[pl.ds(pl.multiple_of(c * N + s * DUMP, 8), DUMP)])

    @pl.when(s == 15)
    def _():
        pltpu.sync_copy(acc.at[pl.ds(16 * DUMP, 16)],
                        out.at[pl.ds(pl.multiple_of(c * N, 8) + 16 * DUMP, 16)])


def _sc_scatter(tab, idxall):
    mesh = plsc.VectorSubcoreMesh(core_axis_name="c", subcore_axis_name="s")
    return pl.kernel(
        _sc_body,
        mesh=mesh,
        out_type=jax.ShapeDtypeStruct((2 * N, D), jnp.float32),
        scratch_types=(
            [pltpu.VMEM((2 * SUP, CW), jnp.int32)] * 2
            + [pltpu.VMEM((CW, D), jnp.float32)] * W
            + [pltpu.VMEM_SHARED((N_PAD, D), jnp.float32)]
            + [pltpu.SemaphoreType.DMA] * (2 + 2 * W)
        ),
    )(tab, idxall)


def kernel(x, edge_index, t, W1, b1, mg, mb, W2, b2, lng, lnb):
    src = edge_index[0]
    dst = edge_index[1]
    sv = src.reshape(16, NSTAGE, SUP, CW)
    dv = dst.reshape(16, NSTAGE, SUP, CW)
    idxall = jnp.stack([
        jnp.concatenate([sv, dv], axis=2),
        jnp.concatenate([sv + N, dv], axis=2),
    ]).reshape(-1, CW)
    del sv, dv
    ts = t.reshape(L, 1, 1)

    pq = _tab0(x, ts[0])

    h = None
    zin = x
    for l in range(L):
        sums = _sc_scatter(pq.reshape(2 * N, D), idxall).reshape(2, N, D)
        final = l == L - 1
        g_idx = 0 if final else l + 1
        tn = ts[0] if final else ts[l + 1]
        outs = _dense(sums, zin, x if h is None else h,
                      W1[l], b1[l].reshape(1, H), mg[l].reshape(1, H),
                      mb[l].reshape(1, H), W2[l], b2[l].reshape(1, D),
                      lng[g_idx].reshape(1, D), lnb[g_idx].reshape(1, D),
                      tn, has_resid=l > 0, final=final)
        if final:
            return outs[0]
        h, zin, pq = outs
